# baseline (speedup 1.0000x reference)
import functools

import jax
import jax.numpy as jnp
from jax import lax
from jax.experimental import pallas as pl
from jax.experimental.pallas import tpu as pltpu
from jax.experimental.pallas import tpu_sc as plsc

_N = 10000
_E = 320000
_T = 12
_DIN = 128
_DSP = 64
_DTM = 64
_B = 32

_NC = 2
_NS = 16
_NW = _NC * _NS
_CHUNK = 128
_EP = 327680
_EPW = _EP // _NW
_NCHUNK = _EPW // _CHUNK
_NP = 10240
_ROWS_PER_TILE = _NP // _NS
_NB = 2
_NGRP = _NCHUNK // _NB
_ZR = 160

_sc_mesh = functools.partial(
    plsc.VectorSubcoreMesh, core_axis_name="c", subcore_axis_name="s")


@functools.partial(
    pl.kernel,
    out_type=(
        jax.ShapeDtypeStruct((_NC, _NP, 8), jnp.float32),
        jax.ShapeDtypeStruct((_NC, _NP, 8), jnp.float32),
    ),
    mesh=_sc_mesh(),
    compiler_params=pltpu.CompilerParams(use_tc_tiling_on_sc=False),
    scratch_types=[
        pltpu.VMEM((_NCHUNK, _CHUNK), jnp.int32),
        pltpu.VMEM((_NCHUNK, _CHUNK), jnp.int32),
        pltpu.VMEM((_CHUNK, 8), jnp.float32),
        pltpu.VMEM((_ROWS_PER_TILE, 8), jnp.float32),
        pltpu.VMEM_SHARED((_NP, 8), jnp.float32),
    ],
)
def _sc_degrees(src_hbm, dst_hbm, ones_hbm, zeros_hbm, dsrc_hbm, ddst_hbm,
                src_v, dst_v, ones_v, zero_v, hist_s):
    cid = lax.axis_index("c")
    sid = lax.axis_index("s")
    wid = sid * _NC + cid

    pltpu.sync_copy(ones_hbm, ones_v)
    pltpu.sync_copy(zeros_hbm, zero_v)
    pltpu.sync_copy(src_hbm.at[wid], src_v)
    pltpu.sync_copy(dst_hbm.at[wid], dst_v)

    row0 = sid * _ROWS_PER_TILE
    for idx_ref, out_ref in ((src_v, dsrc_hbm), (dst_v, ddst_hbm)):
        pltpu.sync_copy(zero_v, hist_s.at[pl.ds(row0, _ROWS_PER_TILE)])
        plsc.subcore_barrier()

        def body(c, _, idx_ref=idx_ref):
            pltpu.sync_copy(ones_v, hist_s.at[idx_ref.at[c]], add=True)
            return 0
        lax.fori_loop(0, _NCHUNK, body, 0)

        plsc.subcore_barrier()
        pltpu.sync_copy(hist_s.at[pl.ds(row0, _ROWS_PER_TILE)],
                        out_ref.at[cid, pl.ds(row0, _ROWS_PER_TILE)])


@functools.partial(
    pl.kernel,
    out_type=jax.ShapeDtypeStruct((_NC, _T, _NP, _DSP), jnp.float32),
    mesh=_sc_mesh(),
    compiler_params=pltpu.CompilerParams(use_tc_tiling_on_sc=False),
    scratch_types=[
        pltpu.VMEM((_NCHUNK, _CHUNK), jnp.int32),
        pltpu.VMEM((_NCHUNK, _CHUNK), jnp.int32),
        pltpu.VMEM((2, _NB, _CHUNK), jnp.int32),
        pltpu.VMEM((2, _NB, _CHUNK, _DSP), jnp.float32),
        pltpu.VMEM((_ZR, _DSP), jnp.float32),
        pltpu.VMEM_SHARED((_NP, _DSP), jnp.float32),
        pltpu.SemaphoreType.DMA,
        pltpu.SemaphoreType.DMA,
    ],
)
def _sc_edge_scatter(h_hbm, src_hbm, dst_hbm, zeros_hbm, out_hbm,
                     src_v, dst_v, idx_v, rows_v, zero_v, agg_s, gsem, ssem):
    cid = lax.axis_index("c")
    sid = lax.axis_index("s")
    wid = sid * _NC + cid
    row0 = sid * _ROWS_PER_TILE

    pltpu.sync_copy(zeros_hbm, zero_v)
    pltpu.sync_copy(src_hbm.at[wid], src_v)
    pltpu.sync_copy(dst_hbm.at[wid], dst_v)

    def zero_slice():
        for z in range(_ROWS_PER_TILE // _ZR):
            pltpu.sync_copy(zero_v, agg_s.at[pl.ds(row0 + z * _ZR, _ZR)])

    def fire_gathers(bank, g, base):
        for b in range(_NB):
            c = g * _NB + b
            for j in range(_CHUNK // 16):
                idx_v[bank, b, pl.ds(j * 16, 16)] = (
                    src_v[c, pl.ds(j * 16, 16)] + base)
            pltpu.async_copy(h_hbm.at[idx_v.at[bank, b]],
                             rows_v.at[bank, b], gsem)

    def wait_gathers(bank):
        for b in range(_NB):
            pltpu.make_async_copy(h_hbm.at[idx_v.at[bank, b]],
                                  rows_v.at[bank, b], gsem).wait()

    def fire_scatters(bank, g):
        for b in range(_NB):
            c = g * _NB + b
            pltpu.async_copy(rows_v.at[bank, b], agg_s.at[dst_v.at[c]],
                             ssem, add=True)

    def drain_scatters(bank):
        for b in range(_NB):
            pltpu.make_async_copy(rows_v.at[bank, b],
                                  agg_s.at[dst_v.at[0]], ssem).wait()

    for t in range(_T):
        base = t * _NP
        zero_slice()
        plsc.subcore_barrier()

        fire_gathers(0, 0, base)

        def body(i, _):
            wait_gathers(0)
            fire_scatters(0, 2 * i)

            @pl.when(i > 0)
            def _():
                drain_scatters(1)
            fire_gathers(1, 2 * i + 1, base)
            wait_gathers(1)
            fire_scatters(1, 2 * i + 1)
            drain_scatters(0)

            @pl.when(i < _NGRP // 2 - 1)
            def _():
                fire_gathers(0, 2 * i + 2, base)
            return 0
        lax.fori_loop(0, _NGRP // 2, body, 0)

        drain_scatters(1)
        plsc.subcore_barrier()
        pltpu.sync_copy(agg_s.at[pl.ds(row0, _ROWS_PER_TILE)],
                        out_hbm.at[cid, t, pl.ds(row0, _ROWS_PER_TILE)])


_BN = 400
_GRID = _N // _BN


def _norm_from(deg_ref):
    d = deg_ref[0, :, 0:1] + deg_ref[1, :, 0:1]
    return jax.lax.rsqrt(jnp.maximum(d, 1.0))


def _gelu(x):
    return x * 0.5 * (1.0 + lax.erf(x * 0.7071067811865476))


def _pre1_body(x_ref, w_ref, dsrc_ref, o_ref):
    w = w_ref[...]
    ns = _norm_from(dsrc_ref)
    for t in range(_T):
        xt = x_ref[:, t, :]
        o_ref[t] = jnp.dot(xt, w, preferred_element_type=jnp.float32) * ns


def _pre1(traffic_h, W_g1, dsrc):
    return pl.pallas_call(
        _pre1_body,
        grid=(_GRID,),
        in_specs=[
            pl.BlockSpec((_BN, _T, _DIN), lambda i: (i, 0, 0)),
            pl.BlockSpec((_DIN, _DSP), lambda i: (0, 0)),
            pl.BlockSpec((_NC, _BN, 8), lambda i: (0, i, 0)),
        ],
        out_specs=pl.BlockSpec((_T, _BN, _DSP), lambda i: (0, i, 0)),
        out_shape=jax.ShapeDtypeStruct((_T, _NP, _DSP), jnp.float32),
    )(traffic_h, W_g1, dsrc)


def _mid_body(agg_ref, ddst_ref, b1_ref, w2_ref, dsrc_ref, o_ref):
    nd = _norm_from(ddst_ref)[None]
    ns = _norm_from(dsrc_ref)
    b1 = b1_ref[...]
    w2 = w2_ref[...]
    a = agg_ref[0] + agg_ref[1]
    h1 = _gelu(a * nd + b1[None])
    for t in range(_T):
        o_ref[t] = jnp.dot(h1[t], w2, preferred_element_type=jnp.float32) * ns


def _mid(agg1, ddst, b_g1, W_g2, dsrc):
    return pl.pallas_call(
        _mid_body,
        grid=(_GRID,),
        in_specs=[
            pl.BlockSpec((_NC, _T, _BN, _DSP), lambda i: (0, 0, i, 0)),
            pl.BlockSpec((_NC, _BN, 8), lambda i: (0, i, 0)),
            pl.BlockSpec((1, _DSP), lambda i: (0, 0)),
            pl.BlockSpec((_DSP, _DSP), lambda i: (0, 0)),
            pl.BlockSpec((_NC, _BN, 8), lambda i: (0, i, 0)),
        ],
        out_specs=pl.BlockSpec((_T, _BN, _DSP), lambda i: (0, i, 0)),
        out_shape=jax.ShapeDtypeStruct((_T, _NP, _DSP), jnp.float32),
    )(agg1, ddst, b_g1, W_g2, dsrc)


def _post_body(agg_ref, ddst_ref, b2_ref, bidx_ref, weeks_ref, minutes_ref,
               wtab_ref, mtab_ref, hs_ref, wk_ref, mn_ref, cnt_ref):
    i = pl.program_id(0)

    @pl.when(i == 0)
    def _init():
        hs_ref[...] = jnp.zeros_like(hs_ref)
        cnt_ref[...] = jnp.zeros_like(cnt_ref)

    nd = _norm_from(ddst_ref)[None]
    b2 = b2_ref[...]
    a = agg_ref[0] + agg_ref[1]
    h2 = _gelu(a * nd + b2[None])

    bidx = bidx_ref[...][:, 0]
    onehot = (bidx[None, :] ==
              lax.broadcasted_iota(jnp.int32, (_B, _BN), 0)
              ).astype(jnp.float32)
    cnt_ref[:, 0:1] += jnp.sum(onehot, axis=1, keepdims=True)
    for t in range(_T):
        hs_ref[:, t, :] += jnp.dot(onehot, h2[t],
                                   preferred_element_type=jnp.float32)

    @pl.when(i == _GRID - 1)
    def _finish():
        cnt = jnp.maximum(cnt_ref[:, 0:1], 1.0)
        hs_ref[...] = hs_ref[...] / cnt[:, None]
        wk = weeks_ref[...]
        mi = minutes_ref[...]
        ohw = (wk[:, :, None] ==
               lax.broadcasted_iota(jnp.int32, (_B, _T, 7), 2)
               ).astype(jnp.float32).reshape(_B * _T, 7)
        ohm = (mi[:, :, None] ==
               lax.broadcasted_iota(jnp.int32, (_B, _T, 288), 2)
               ).astype(jnp.float32).reshape(_B * _T, 288)
        wk_ref[...] = jnp.dot(ohw, wtab_ref[...],
                              preferred_element_type=jnp.float32
                              ).reshape(_B, _T, 3)
        mn_ref[...] = jnp.dot(ohm, mtab_ref[...],
                              preferred_element_type=jnp.float32
                              ).reshape(_B, _T, 32)


def _post(agg2, ddst, b_g2, bidx, weeks, minutes, week_table, minute_table):
    return pl.pallas_call(
        _post_body,
        grid=(_GRID,),
        in_specs=[
            pl.BlockSpec((_NC, _T, _BN, _DSP), lambda i: (0, 0, i, 0)),
            pl.BlockSpec((_NC, _BN, 8), lambda i: (0, i, 0)),
            pl.BlockSpec((1, _DSP), lambda i: (0, 0)),
            pl.BlockSpec((_BN, 1), lambda i: (i, 0)),
            pl.BlockSpec((_B, _T), lambda i: (0, 0)),
            pl.BlockSpec((_B, _T), lambda i: (0, 0)),
            pl.BlockSpec((7, 3), lambda i: (0, 0)),
            pl.BlockSpec((288, 32), lambda i: (0, 0)),
        ],
        out_specs=[
            pl.BlockSpec((_B, _T, _DSP), lambda i: (0, 0, 0)),
            pl.BlockSpec((_B, _T, 3), lambda i: (0, 0, 0)),
            pl.BlockSpec((_B, _T, 32), lambda i: (0, 0, 0)),
        ],
        out_shape=[
            jax.ShapeDtypeStruct((_B, _T, _DSP), jnp.float32),
            jax.ShapeDtypeStruct((_B, _T, 3), jnp.float32),
            jax.ShapeDtypeStruct((_B, _T, 32), jnp.float32),
        ],
        scratch_shapes=[pltpu.VMEM((_B, 128), jnp.float32)],
    )(agg2, ddst, b_g2, bidx, weeks, minutes, week_table, minute_table)


def _lstm_body(x_ref, wih_ref, whh_ref, bih_ref, bhh_ref, o_ref):
    x = x_ref[...]
    wih = wih_ref[...]
    whh = whh_ref[...]
    b = bih_ref[...] + bhh_ref[...]
    xw = jnp.dot(x.reshape(_BN * _T, _DIN), wih,
                 preferred_element_type=jnp.float32
                 ).reshape(_BN, _T, 4 * _DTM)
    h = jnp.zeros((_BN, _DTM), jnp.float32)
    c = jnp.zeros((_BN, _DTM), jnp.float32)
    acc = jnp.zeros((_BN, _DTM), jnp.float32)
    for t in range(_T):
        g = xw[:, t, :] + jnp.dot(h, whh,
                                  preferred_element_type=jnp.float32) + b
        ig = jax.nn.sigmoid(g[:, 0:_DTM])
        fg = jax.nn.sigmoid(g[:, _DTM:2 * _DTM])
        gg = jnp.tanh(g[:, 2 * _DTM:3 * _DTM])
        og = jax.nn.sigmoid(g[:, 3 * _DTM:4 * _DTM])
        c = fg * c + ig * gg
        h = og * jnp.tanh(c)
        acc = acc + h
    o_ref[...] = acc * (1.0 / _T)


def _lstm(traffic_h, W_ih, W_hh, b_ih, b_hh):
    return pl.pallas_call(
        _lstm_body,
        grid=(_GRID,),
        in_specs=[
            pl.BlockSpec((_BN, _T, _DIN), lambda i: (i, 0, 0)),
            pl.BlockSpec((_DIN, 4 * _DTM), lambda i: (0, 0)),
            pl.BlockSpec((_DTM, 4 * _DTM), lambda i: (0, 0)),
            pl.BlockSpec((1, 4 * _DTM), lambda i: (0, 0)),
            pl.BlockSpec((1, 4 * _DTM), lambda i: (0, 0)),
        ],
        out_specs=pl.BlockSpec((_BN, _DTM), lambda i: (i, 0)),
        out_shape=jax.ShapeDtypeStruct((_N, _DTM), jnp.float32),
    )(traffic_h, W_ih, W_hh, b_ih, b_hh)


def kernel(weeks, minutes, global_spatial_idx, edge_index, traffic_h,
           local_batch_idx, local_spatial_idx, local_spatial_feature,
           W_g1, b_g1, W_g2, b_g2, W_ih, W_hh, b_ih, b_hh,
           week_table, minute_table):
    pad = (_N + jnp.arange(_EP - _E, dtype=jnp.int32) % (_NP - _N))
    src3d = jnp.concatenate([edge_index[0], pad]).reshape(_NW, _NCHUNK, _CHUNK)
    dst3d = jnp.concatenate([edge_index[1], pad]).reshape(_NW, _NCHUNK, _CHUNK)

    ones8 = jnp.ones((_CHUNK, 8), jnp.float32)
    zeros8 = jnp.zeros((_ROWS_PER_TILE, 8), jnp.float32)
    zeros64 = jnp.zeros((_ZR, _DSP), jnp.float32)
    dsrc, ddst = _sc_degrees(src3d, dst3d, ones8, zeros8)

    h1pre = _pre1(traffic_h, W_g1, dsrc)
    agg1 = _sc_edge_scatter(h1pre.reshape(_T * _NP, _DSP), src3d, dst3d, zeros64)
    h2pre = _mid(agg1, ddst, b_g1.reshape(1, _DSP), W_g2, dsrc)
    agg2 = _sc_edge_scatter(h2pre.reshape(_T * _NP, _DSP), src3d, dst3d, zeros64)
    hs, week_emb, minute_emb = _post(
        agg2, ddst, b_g2.reshape(1, _DSP), local_batch_idx.reshape(_N, 1),
        weeks, minutes, week_table, minute_table)

    ht = _lstm(traffic_h, W_ih, W_hh,
               b_ih.reshape(1, 4 * _DTM), b_hh.reshape(1, 4 * _DTM))

    return hs, ht, week_emb, minute_emb


# speedup vs baseline: 1.0898x; 1.0898x over previous
import functools

import jax
import jax.numpy as jnp
from jax import lax
from jax.experimental import pallas as pl
from jax.experimental.pallas import tpu as pltpu
from jax.experimental.pallas import tpu_sc as plsc

_N = 10000
_E = 320000
_T = 12
_DIN = 128
_DSP = 64
_DTM = 64
_B = 32

_NC = 2
_NS = 16
_NW = _NC * _NS
_CHUNK = 128
_EP = 327680
_EPW = _EP // _NW
_NCHUNK = _EPW // _CHUNK
_NP = 10240
_ROWS_PER_TILE = _NP // _NS
_NB = 4
_NGRP = _NCHUNK // _NB
_ZR = 80

_sc_mesh = functools.partial(
    plsc.VectorSubcoreMesh, core_axis_name="c", subcore_axis_name="s")


@functools.partial(
    pl.kernel,
    out_type=(
        jax.ShapeDtypeStruct((_NC, _NP, 8), jnp.float32),
        jax.ShapeDtypeStruct((_NC, _NP, 8), jnp.float32),
    ),
    mesh=_sc_mesh(),
    compiler_params=pltpu.CompilerParams(use_tc_tiling_on_sc=False),
    scratch_types=[
        pltpu.VMEM((_NCHUNK, _CHUNK), jnp.int32),
        pltpu.VMEM((_CHUNK, 8), jnp.float32),
        pltpu.VMEM((_ZR, 8), jnp.float32),
        pltpu.VMEM_SHARED((_NP, 8), jnp.float32),
    ],
)
def _sc_degrees(src_hbm, dst_hbm, ones_hbm, zeros_hbm, dsrc_hbm, ddst_hbm,
                e_v, ones_v, zero_v, hist_s):
    cid = lax.axis_index("c")
    sid = lax.axis_index("s")
    wid = sid * _NC + cid

    pltpu.sync_copy(ones_hbm, ones_v)
    pltpu.sync_copy(zeros_hbm, zero_v)

    row0 = sid * _ROWS_PER_TILE
    for e_hbm, out_ref in ((src_hbm, dsrc_hbm), (dst_hbm, ddst_hbm)):
        pltpu.sync_copy(e_hbm.at[wid], e_v)
        for z in range(_ROWS_PER_TILE // _ZR):
            pltpu.sync_copy(zero_v, hist_s.at[pl.ds(row0 + z * _ZR, _ZR)])
        plsc.subcore_barrier()

        def body(c, _):
            pltpu.sync_copy(ones_v, hist_s.at[e_v.at[c]], add=True)
            return 0
        lax.fori_loop(0, _NCHUNK, body, 0)

        plsc.subcore_barrier()
        pltpu.sync_copy(hist_s.at[pl.ds(row0, _ROWS_PER_TILE)],
                        out_ref.at[cid, pl.ds(row0, _ROWS_PER_TILE)])


@functools.partial(
    pl.kernel,
    out_type=jax.ShapeDtypeStruct((_NC, _T, _NP, _DSP), jnp.float32),
    mesh=_sc_mesh(),
    compiler_params=pltpu.CompilerParams(use_tc_tiling_on_sc=False),
    scratch_types=[
        pltpu.VMEM((_NCHUNK, _CHUNK), jnp.int32),
        pltpu.VMEM((_NCHUNK, _CHUNK), jnp.int32),
        pltpu.VMEM((2, _NB, _CHUNK), jnp.int32),
        pltpu.VMEM((2, _NB, _CHUNK, _DSP), jnp.float32),
        pltpu.VMEM_SHARED((_NP, _DSP), jnp.float32),
        pltpu.SemaphoreType.DMA,
        pltpu.SemaphoreType.DMA,
    ],
)
def _sc_edge_scatter(h_hbm, src_hbm, dst_hbm, zeros_hbm, out_hbm,
                     src_v, dst_v, idx_v, rows_v, agg_s, gsem, ssem):
    cid = lax.axis_index("c")
    sid = lax.axis_index("s")
    wid = sid * _NC + cid
    row0 = sid * _ROWS_PER_TILE

    pltpu.sync_copy(src_hbm.at[wid], src_v)
    pltpu.sync_copy(dst_hbm.at[wid], dst_v)

    def zero_slice():
        pltpu.sync_copy(zeros_hbm.at[pl.ds(row0, _ROWS_PER_TILE)],
                        agg_s.at[pl.ds(row0, _ROWS_PER_TILE)])

    def fire_gathers(bank, g, base):
        for b in range(_NB):
            c = g * _NB + b
            for j in range(_CHUNK // 16):
                idx_v[bank, b, pl.ds(j * 16, 16)] = (
                    src_v[c, pl.ds(j * 16, 16)] + base)
            pltpu.async_copy(h_hbm.at[idx_v.at[bank, b]],
                             rows_v.at[bank, b], gsem)

    def wait_gathers(bank):
        for b in range(_NB):
            pltpu.make_async_copy(h_hbm.at[idx_v.at[bank, b]],
                                  rows_v.at[bank, b], gsem).wait()

    def fire_scatters(bank, g):
        for b in range(_NB):
            c = g * _NB + b
            pltpu.async_copy(rows_v.at[bank, b], agg_s.at[dst_v.at[c]],
                             ssem, add=True)

    def drain_scatters(bank):
        for b in range(_NB):
            pltpu.make_async_copy(rows_v.at[bank, b],
                                  agg_s.at[dst_v.at[0]], ssem).wait()

    for t in range(_T):
        base = t * _NP
        zero_slice()
        plsc.subcore_barrier()

        fire_gathers(0, 0, base)

        def body(i, _):
            wait_gathers(0)
            fire_scatters(0, 2 * i)

            @pl.when(i > 0)
            def _():
                drain_scatters(1)
            fire_gathers(1, 2 * i + 1, base)
            wait_gathers(1)
            fire_scatters(1, 2 * i + 1)
            drain_scatters(0)

            @pl.when(i < _NGRP // 2 - 1)
            def _():
                fire_gathers(0, 2 * i + 2, base)
            return 0
        lax.fori_loop(0, _NGRP // 2, body, 0)

        drain_scatters(1)
        plsc.subcore_barrier()
        pltpu.sync_copy(agg_s.at[pl.ds(row0, _ROWS_PER_TILE)],
                        out_hbm.at[cid, t, pl.ds(row0, _ROWS_PER_TILE)])


_BN = 400
_GRID = _N // _BN


def _norm_from(deg_ref):
    d = deg_ref[0, :, 0:1] + deg_ref[1, :, 0:1]
    return jax.lax.rsqrt(jnp.maximum(d, 1.0))


def _gelu(x):
    return x * 0.5 * (1.0 + lax.erf(x * 0.7071067811865476))


def _pre1_body(x_ref, w_ref, dsrc_ref, o_ref):
    w = w_ref[...]
    ns = _norm_from(dsrc_ref)
    for t in range(_T):
        xt = x_ref[:, t, :]
        o_ref[t] = jnp.dot(xt, w, preferred_element_type=jnp.float32) * ns


def _pre1(traffic_h, W_g1, dsrc):
    return pl.pallas_call(
        _pre1_body,
        grid=(_GRID,),
        in_specs=[
            pl.BlockSpec((_BN, _T, _DIN), lambda i: (i, 0, 0)),
            pl.BlockSpec((_DIN, _DSP), lambda i: (0, 0)),
            pl.BlockSpec((_NC, _BN, 8), lambda i: (0, i, 0)),
        ],
        out_specs=pl.BlockSpec((_T, _BN, _DSP), lambda i: (0, i, 0)),
        out_shape=jax.ShapeDtypeStruct((_T, _NP, _DSP), jnp.float32),
    )(traffic_h, W_g1, dsrc)


def _mid_body(agg_ref, ddst_ref, b1_ref, w2_ref, dsrc_ref, o_ref):
    nd = _norm_from(ddst_ref)[None]
    ns = _norm_from(dsrc_ref)
    b1 = b1_ref[...]
    w2 = w2_ref[...]
    a = agg_ref[0] + agg_ref[1]
    h1 = _gelu(a * nd + b1[None])
    for t in range(_T):
        o_ref[t] = jnp.dot(h1[t], w2, preferred_element_type=jnp.float32) * ns


def _mid(agg1, ddst, b_g1, W_g2, dsrc):
    return pl.pallas_call(
        _mid_body,
        grid=(_GRID,),
        in_specs=[
            pl.BlockSpec((_NC, _T, _BN, _DSP), lambda i: (0, 0, i, 0)),
            pl.BlockSpec((_NC, _BN, 8), lambda i: (0, i, 0)),
            pl.BlockSpec((1, _DSP), lambda i: (0, 0)),
            pl.BlockSpec((_DSP, _DSP), lambda i: (0, 0)),
            pl.BlockSpec((_NC, _BN, 8), lambda i: (0, i, 0)),
        ],
        out_specs=pl.BlockSpec((_T, _BN, _DSP), lambda i: (0, i, 0)),
        out_shape=jax.ShapeDtypeStruct((_T, _NP, _DSP), jnp.float32),
    )(agg1, ddst, b_g1, W_g2, dsrc)


def _post_body(agg_ref, ddst_ref, b2_ref, bidx_ref, weeks_ref, minutes_ref,
               wtab_ref, mtab_ref, hs_ref, wk_ref, mn_ref, cnt_ref):
    i = pl.program_id(0)

    @pl.when(i == 0)
    def _init():
        hs_ref[...] = jnp.zeros_like(hs_ref)
        cnt_ref[...] = jnp.zeros_like(cnt_ref)

    nd = _norm_from(ddst_ref)[None]
    b2 = b2_ref[...]
    a = agg_ref[0] + agg_ref[1]
    h2 = _gelu(a * nd + b2[None])

    bidx = bidx_ref[...][:, 0]
    onehot = (bidx[None, :] ==
              lax.broadcasted_iota(jnp.int32, (_B, _BN), 0)
              ).astype(jnp.float32)
    cnt_ref[:, 0:1] += jnp.sum(onehot, axis=1, keepdims=True)
    for t in range(_T):
        hs_ref[:, t, :] += jnp.dot(onehot, h2[t],
                                   preferred_element_type=jnp.float32)

    @pl.when(i == _GRID - 1)
    def _finish():
        cnt = jnp.maximum(cnt_ref[:, 0:1], 1.0)
        hs_ref[...] = hs_ref[...] / cnt[:, None]
        wk = weeks_ref[...]
        mi = minutes_ref[...]
        ohw = (wk[:, :, None] ==
               lax.broadcasted_iota(jnp.int32, (_B, _T, 7), 2)
               ).astype(jnp.float32).reshape(_B * _T, 7)
        ohm = (mi[:, :, None] ==
               lax.broadcasted_iota(jnp.int32, (_B, _T, 288), 2)
               ).astype(jnp.float32).reshape(_B * _T, 288)
        wk_ref[...] = jnp.dot(ohw, wtab_ref[...],
                              preferred_element_type=jnp.float32
                              ).reshape(_B, _T, 3)
        mn_ref[...] = jnp.dot(ohm, mtab_ref[...],
                              preferred_element_type=jnp.float32
                              ).reshape(_B, _T, 32)


def _post(agg2, ddst, b_g2, bidx, weeks, minutes, week_table, minute_table):
    return pl.pallas_call(
        _post_body,
        grid=(_GRID,),
        in_specs=[
            pl.BlockSpec((_NC, _T, _BN, _DSP), lambda i: (0, 0, i, 0)),
            pl.BlockSpec((_NC, _BN, 8), lambda i: (0, i, 0)),
            pl.BlockSpec((1, _DSP), lambda i: (0, 0)),
            pl.BlockSpec((_BN, 1), lambda i: (i, 0)),
            pl.BlockSpec((_B, _T), lambda i: (0, 0)),
            pl.BlockSpec((_B, _T), lambda i: (0, 0)),
            pl.BlockSpec((7, 3), lambda i: (0, 0)),
            pl.BlockSpec((288, 32), lambda i: (0, 0)),
        ],
        out_specs=[
            pl.BlockSpec((_B, _T, _DSP), lambda i: (0, 0, 0)),
            pl.BlockSpec((_B, _T, 3), lambda i: (0, 0, 0)),
            pl.BlockSpec((_B, _T, 32), lambda i: (0, 0, 0)),
        ],
        out_shape=[
            jax.ShapeDtypeStruct((_B, _T, _DSP), jnp.float32),
            jax.ShapeDtypeStruct((_B, _T, 3), jnp.float32),
            jax.ShapeDtypeStruct((_B, _T, 32), jnp.float32),
        ],
        scratch_shapes=[pltpu.VMEM((_B, 128), jnp.float32)],
    )(agg2, ddst, b_g2, bidx, weeks, minutes, week_table, minute_table)


def _lstm_body(x_ref, wih_ref, whh_ref, bih_ref, bhh_ref, o_ref):
    x = x_ref[...]
    wih = wih_ref[...]
    whh = whh_ref[...]
    b = bih_ref[...] + bhh_ref[...]
    xw = jnp.dot(x.reshape(_BN * _T, _DIN), wih,
                 preferred_element_type=jnp.float32
                 ).reshape(_BN, _T, 4 * _DTM)
    h = jnp.zeros((_BN, _DTM), jnp.float32)
    c = jnp.zeros((_BN, _DTM), jnp.float32)
    acc = jnp.zeros((_BN, _DTM), jnp.float32)
    for t in range(_T):
        g = xw[:, t, :] + jnp.dot(h, whh,
                                  preferred_element_type=jnp.float32) + b
        ig = jax.nn.sigmoid(g[:, 0:_DTM])
        fg = jax.nn.sigmoid(g[:, _DTM:2 * _DTM])
        gg = jnp.tanh(g[:, 2 * _DTM:3 * _DTM])
        og = jax.nn.sigmoid(g[:, 3 * _DTM:4 * _DTM])
        c = fg * c + ig * gg
        h = og * jnp.tanh(c)
        acc = acc + h
    o_ref[...] = acc * (1.0 / _T)


def _lstm(traffic_h, W_ih, W_hh, b_ih, b_hh):
    return pl.pallas_call(
        _lstm_body,
        grid=(_GRID,),
        in_specs=[
            pl.BlockSpec((_BN, _T, _DIN), lambda i: (i, 0, 0)),
            pl.BlockSpec((_DIN, 4 * _DTM), lambda i: (0, 0)),
            pl.BlockSpec((_DTM, 4 * _DTM), lambda i: (0, 0)),
            pl.BlockSpec((1, 4 * _DTM), lambda i: (0, 0)),
            pl.BlockSpec((1, 4 * _DTM), lambda i: (0, 0)),
        ],
        out_specs=pl.BlockSpec((_BN, _DTM), lambda i: (i, 0)),
        out_shape=jax.ShapeDtypeStruct((_N, _DTM), jnp.float32),
    )(traffic_h, W_ih, W_hh, b_ih, b_hh)


def kernel(weeks, minutes, global_spatial_idx, edge_index, traffic_h,
           local_batch_idx, local_spatial_idx, local_spatial_feature,
           W_g1, b_g1, W_g2, b_g2, W_ih, W_hh, b_ih, b_hh,
           week_table, minute_table):
    pad = (_N + jnp.arange(_EP - _E, dtype=jnp.int32) % (_NP - _N))
    src3d = jnp.concatenate([edge_index[0], pad]).reshape(_NW, _NCHUNK, _CHUNK)
    dst3d = jnp.concatenate([edge_index[1], pad]).reshape(_NW, _NCHUNK, _CHUNK)

    ones8 = jnp.ones((_CHUNK, 8), jnp.float32)
    zeros8 = jnp.zeros((_ZR, 8), jnp.float32)
    zeros64 = jnp.zeros((_NP, _DSP), jnp.float32)
    dsrc, ddst = _sc_degrees(src3d, dst3d, ones8, zeros8)

    h1pre = _pre1(traffic_h, W_g1, dsrc)
    agg1 = _sc_edge_scatter(h1pre.reshape(_T * _NP, _DSP), src3d, dst3d, zeros64)
    h2pre = _mid(agg1, ddst, b_g1.reshape(1, _DSP), W_g2, dsrc)
    agg2 = _sc_edge_scatter(h2pre.reshape(_T * _NP, _DSP), src3d, dst3d, zeros64)
    hs, week_emb, minute_emb = _post(
        agg2, ddst, b_g2.reshape(1, _DSP), local_batch_idx.reshape(_N, 1),
        weeks, minutes, week_table, minute_table)

    ht = _lstm(traffic_h, W_ih, W_hh,
               b_ih.reshape(1, 4 * _DTM), b_hh.reshape(1, 4 * _DTM))

    return hs, ht, week_emb, minute_emb


# speedup vs baseline: 1.0900x; 1.0002x over previous
import functools

import jax
import jax.numpy as jnp
from jax import lax
from jax.experimental import pallas as pl
from jax.experimental.pallas import tpu as pltpu
from jax.experimental.pallas import tpu_sc as plsc

_N = 10000
_E = 320000
_T = 12
_DIN = 128
_DSP = 64
_DTM = 64
_B = 32

_NC = 2
_NS = 16
_NW = _NC * _NS
_CHUNK = 128
_EP = 327680
_EPW = _EP // _NW
_NCHUNK = _EPW // _CHUNK
_NP = 10240
_ROWS_PER_TILE = _NP // _NS
_NB = 4
_NGRP = _NCHUNK // _NB
_ZR = 80

_sc_mesh = functools.partial(
    plsc.VectorSubcoreMesh, core_axis_name="c", subcore_axis_name="s")


@functools.partial(
    pl.kernel,
    out_type=(
        jax.ShapeDtypeStruct((_NC, _NP, 8), jnp.float32),
        jax.ShapeDtypeStruct((_NC, _NP, 8), jnp.float32),
    ),
    mesh=_sc_mesh(),
    compiler_params=pltpu.CompilerParams(use_tc_tiling_on_sc=False),
    scratch_types=[
        pltpu.VMEM((_NCHUNK, _CHUNK), jnp.int32),
        pltpu.VMEM((_CHUNK, 8), jnp.float32),
        pltpu.VMEM((_ZR, 8), jnp.float32),
        pltpu.VMEM_SHARED((_NP, 8), jnp.float32),
    ],
)
def _sc_degrees(src_hbm, dst_hbm, ones_hbm, zeros_hbm, dsrc_hbm, ddst_hbm,
                e_v, ones_v, zero_v, hist_s):
    cid = lax.axis_index("c")
    sid = lax.axis_index("s")
    wid = sid * _NC + cid

    pltpu.sync_copy(ones_hbm, ones_v)
    pltpu.sync_copy(zeros_hbm, zero_v)

    row0 = sid * _ROWS_PER_TILE
    for e_hbm, out_ref in ((src_hbm, dsrc_hbm), (dst_hbm, ddst_hbm)):
        pltpu.sync_copy(e_hbm.at[wid], e_v)
        for z in range(_ROWS_PER_TILE // _ZR):
            pltpu.sync_copy(zero_v, hist_s.at[pl.ds(row0 + z * _ZR, _ZR)])
        plsc.subcore_barrier()

        def body(c, _):
            pltpu.sync_copy(ones_v, hist_s.at[e_v.at[c]], add=True)
            return 0
        lax.fori_loop(0, _NCHUNK, body, 0)

        plsc.subcore_barrier()
        pltpu.sync_copy(hist_s.at[pl.ds(row0, _ROWS_PER_TILE)],
                        out_ref.at[cid, pl.ds(row0, _ROWS_PER_TILE)])


@functools.partial(
    pl.kernel,
    out_type=jax.ShapeDtypeStruct((_NC, _T, _NP, _DSP), jnp.float32),
    mesh=_sc_mesh(),
    compiler_params=pltpu.CompilerParams(use_tc_tiling_on_sc=False),
    scratch_types=[
        pltpu.VMEM((_NCHUNK, _CHUNK), jnp.int32),
        pltpu.VMEM((_NCHUNK, _CHUNK), jnp.int32),
        pltpu.VMEM((2, _NB, _CHUNK), jnp.int32),
        pltpu.VMEM((2, _NB, _CHUNK, _DSP), jnp.float32),
        pltpu.VMEM_SHARED((_NP, _DSP), jnp.float32),
        pltpu.SemaphoreType.DMA,
        pltpu.SemaphoreType.DMA,
    ],
)
def _sc_edge_scatter(h_hbm, src_hbm, dst_hbm, zeros_hbm, out_hbm,
                     src_v, dst_v, idx_v, rows_v, agg_s, gsem, ssem):
    cid = lax.axis_index("c")
    sid = lax.axis_index("s")
    wid = sid * _NC + cid
    row0 = sid * _ROWS_PER_TILE

    pltpu.sync_copy(src_hbm.at[wid], src_v)
    pltpu.sync_copy(dst_hbm.at[wid], dst_v)

    def zero_slice():
        pltpu.sync_copy(zeros_hbm.at[pl.ds(row0, _ROWS_PER_TILE)],
                        agg_s.at[pl.ds(row0, _ROWS_PER_TILE)])

    def fire_gathers(bank, g, base):
        for b in range(_NB):
            c = g * _NB + b
            for j in range(_CHUNK // 16):
                idx_v[bank, b, pl.ds(j * 16, 16)] = (
                    src_v[c, pl.ds(j * 16, 16)] + base)
            pltpu.async_copy(h_hbm.at[idx_v.at[bank, b]],
                             rows_v.at[bank, b], gsem)

    def wait_gathers(bank):
        for b in range(_NB):
            pltpu.make_async_copy(h_hbm.at[idx_v.at[bank, b]],
                                  rows_v.at[bank, b], gsem).wait()

    def fire_scatters(bank, g):
        for b in range(_NB):
            c = g * _NB + b
            pltpu.async_copy(rows_v.at[bank, b], agg_s.at[dst_v.at[c]],
                             ssem, add=True)

    def drain_scatters(bank):
        for b in range(_NB):
            pltpu.make_async_copy(rows_v.at[bank, b],
                                  agg_s.at[dst_v.at[0]], ssem).wait()

    for t in range(_T):
        base = t * _NP
        zero_slice()
        plsc.subcore_barrier()

        fire_gathers(0, 0, base)

        def body(i, _):
            wait_gathers(0)
            fire_scatters(0, 2 * i)

            @pl.when(i > 0)
            def _():
                drain_scatters(1)
            fire_gathers(1, 2 * i + 1, base)
            wait_gathers(1)
            fire_scatters(1, 2 * i + 1)
            drain_scatters(0)

            @pl.when(i < _NGRP // 2 - 1)
            def _():
                fire_gathers(0, 2 * i + 2, base)
            return 0
        lax.fori_loop(0, _NGRP // 2, body, 0)

        drain_scatters(1)
        plsc.subcore_barrier()
        pltpu.sync_copy(agg_s.at[pl.ds(row0, _ROWS_PER_TILE)],
                        out_hbm.at[cid, t, pl.ds(row0, _ROWS_PER_TILE)])


_BN = 400
_GRID = _N // _BN


def _norm_from(deg_ref):
    d = deg_ref[0, :, 0:1] + deg_ref[1, :, 0:1]
    return jax.lax.rsqrt(jnp.maximum(d, 1.0))


def _gelu(x):
    return x * 0.5 * (1.0 + lax.erf(x * 0.7071067811865476))


def _pre1_body(x_ref, w_ref, dsrc_ref, o_ref):
    w = w_ref[...]
    ns = _norm_from(dsrc_ref)
    for t in range(_T):
        xt = x_ref[:, t, :]
        o_ref[t] = jnp.dot(xt, w, preferred_element_type=jnp.float32) * ns


def _pre1(traffic_h, W_g1, dsrc):
    return pl.pallas_call(
        _pre1_body,
        grid=(_GRID,),
        in_specs=[
            pl.BlockSpec((_BN, _T, _DIN), lambda i: (i, 0, 0)),
            pl.BlockSpec((_DIN, _DSP), lambda i: (0, 0)),
            pl.BlockSpec((_NC, _BN, 8), lambda i: (0, i, 0)),
        ],
        out_specs=pl.BlockSpec((_T, _BN, _DSP), lambda i: (0, i, 0)),
        out_shape=jax.ShapeDtypeStruct((_T, _NP, _DSP), jnp.float32),
    )(traffic_h, W_g1, dsrc)


def _mid_body(agg_ref, ddst_ref, b1_ref, w2_ref, dsrc_ref, o_ref):
    nd = _norm_from(ddst_ref)[None]
    ns = _norm_from(dsrc_ref)
    b1 = b1_ref[...]
    w2 = w2_ref[...]
    a = agg_ref[0] + agg_ref[1]
    h1 = _gelu(a * nd + b1[None])
    for t in range(_T):
        o_ref[t] = jnp.dot(h1[t], w2, preferred_element_type=jnp.float32) * ns


def _mid(agg1, ddst, b_g1, W_g2, dsrc):
    return pl.pallas_call(
        _mid_body,
        grid=(_GRID,),
        in_specs=[
            pl.BlockSpec((_NC, _T, _BN, _DSP), lambda i: (0, 0, i, 0)),
            pl.BlockSpec((_NC, _BN, 8), lambda i: (0, i, 0)),
            pl.BlockSpec((1, _DSP), lambda i: (0, 0)),
            pl.BlockSpec((_DSP, _DSP), lambda i: (0, 0)),
            pl.BlockSpec((_NC, _BN, 8), lambda i: (0, i, 0)),
        ],
        out_specs=pl.BlockSpec((_T, _BN, _DSP), lambda i: (0, i, 0)),
        out_shape=jax.ShapeDtypeStruct((_T, _NP, _DSP), jnp.float32),
    )(agg1, ddst, b_g1, W_g2, dsrc)


def _post_body(agg_ref, ddst_ref, b2_ref, bidx_ref, weeks_ref, minutes_ref,
               wtab_ref, mtab_ref, hs_ref, wk_ref, mn_ref, cnt_ref):
    i = pl.program_id(0)

    @pl.when(i == 0)
    def _init():
        hs_ref[...] = jnp.zeros_like(hs_ref)
        cnt_ref[...] = jnp.zeros_like(cnt_ref)

    nd = _norm_from(ddst_ref)[None]
    b2 = b2_ref[...]
    a = agg_ref[0] + agg_ref[1]
    h2 = _gelu(a * nd + b2[None])

    bidx = bidx_ref[...][:, 0]
    onehot = (bidx[None, :] ==
              lax.broadcasted_iota(jnp.int32, (_B, _BN), 0)
              ).astype(jnp.float32)
    cnt_ref[:, 0:1] += jnp.sum(onehot, axis=1, keepdims=True)
    for t in range(_T):
        hs_ref[:, t, :] += jnp.dot(onehot, h2[t],
                                   preferred_element_type=jnp.float32)

    @pl.when(i == _GRID - 1)
    def _finish():
        cnt = jnp.maximum(cnt_ref[:, 0:1], 1.0)
        hs_ref[...] = hs_ref[...] / cnt[:, None]
        wk = weeks_ref[...]
        mi = minutes_ref[...]
        ohw = (wk[:, :, None] ==
               lax.broadcasted_iota(jnp.int32, (_B, _T, 7), 2)
               ).astype(jnp.float32).reshape(_B * _T, 7)
        ohm = (mi[:, :, None] ==
               lax.broadcasted_iota(jnp.int32, (_B, _T, 288), 2)
               ).astype(jnp.float32).reshape(_B * _T, 288)
        wk_ref[...] = jnp.dot(ohw, wtab_ref[...],
                              preferred_element_type=jnp.float32
                              ).reshape(_B, _T, 3)
        mn_ref[...] = jnp.dot(ohm, mtab_ref[...],
                              preferred_element_type=jnp.float32
                              ).reshape(_B, _T, 32)


def _post(agg2, ddst, b_g2, bidx, weeks, minutes, week_table, minute_table):
    return pl.pallas_call(
        _post_body,
        grid=(_GRID,),
        in_specs=[
            pl.BlockSpec((_NC, _T, _BN, _DSP), lambda i: (0, 0, i, 0)),
            pl.BlockSpec((_NC, _BN, 8), lambda i: (0, i, 0)),
            pl.BlockSpec((1, _DSP), lambda i: (0, 0)),
            pl.BlockSpec((_BN, 1), lambda i: (i, 0)),
            pl.BlockSpec((_B, _T), lambda i: (0, 0)),
            pl.BlockSpec((_B, _T), lambda i: (0, 0)),
            pl.BlockSpec((7, 3), lambda i: (0, 0)),
            pl.BlockSpec((288, 32), lambda i: (0, 0)),
        ],
        out_specs=[
            pl.BlockSpec((_B, _T, _DSP), lambda i: (0, 0, 0)),
            pl.BlockSpec((_B, _T, 3), lambda i: (0, 0, 0)),
            pl.BlockSpec((_B, _T, 32), lambda i: (0, 0, 0)),
        ],
        out_shape=[
            jax.ShapeDtypeStruct((_B, _T, _DSP), jnp.float32),
            jax.ShapeDtypeStruct((_B, _T, 3), jnp.float32),
            jax.ShapeDtypeStruct((_B, _T, 32), jnp.float32),
        ],
        scratch_shapes=[pltpu.VMEM((_B, 128), jnp.float32)],
    )(agg2, ddst, b_g2, bidx, weeks, minutes, week_table, minute_table)


_BNL = 1000
_GRIDL = _N // _BNL


def _lstm_body(x_ref, wih_ref, whh_ref, bih_ref, bhh_ref, o_ref):
    x = x_ref[...]
    wih = wih_ref[...]
    whh = whh_ref[...]
    b = bih_ref[...] + bhh_ref[...]
    xw = jnp.dot(x.reshape(_BNL * _T, _DIN), wih,
                 preferred_element_type=jnp.float32
                 ).reshape(_BNL, _T, 4 * _DTM)
    h = jnp.zeros((_BNL, _DTM), jnp.float32)
    c = jnp.zeros((_BNL, _DTM), jnp.float32)
    acc = jnp.zeros((_BNL, _DTM), jnp.float32)
    for t in range(_T):
        g = xw[:, t, :] + jnp.dot(h, whh,
                                  preferred_element_type=jnp.float32) + b
        ig = jax.nn.sigmoid(g[:, 0:_DTM])
        fg = jax.nn.sigmoid(g[:, _DTM:2 * _DTM])
        gg = jnp.tanh(g[:, 2 * _DTM:3 * _DTM])
        og = jax.nn.sigmoid(g[:, 3 * _DTM:4 * _DTM])
        c = fg * c + ig * gg
        h = og * jnp.tanh(c)
        acc = acc + h
    o_ref[...] = acc * (1.0 / _T)


def _lstm(traffic_h, W_ih, W_hh, b_ih, b_hh):
    return pl.pallas_call(
        _lstm_body,
        grid=(_GRIDL,),
        in_specs=[
            pl.BlockSpec((_BNL, _T, _DIN), lambda i: (i, 0, 0)),
            pl.BlockSpec((_DIN, 4 * _DTM), lambda i: (0, 0)),
            pl.BlockSpec((_DTM, 4 * _DTM), lambda i: (0, 0)),
            pl.BlockSpec((1, 4 * _DTM), lambda i: (0, 0)),
            pl.BlockSpec((1, 4 * _DTM), lambda i: (0, 0)),
        ],
        out_specs=pl.BlockSpec((_BNL, _DTM), lambda i: (i, 0)),
        out_shape=jax.ShapeDtypeStruct((_N, _DTM), jnp.float32),
    )(traffic_h, W_ih, W_hh, b_ih, b_hh)


def kernel(weeks, minutes, global_spatial_idx, edge_index, traffic_h,
           local_batch_idx, local_spatial_idx, local_spatial_feature,
           W_g1, b_g1, W_g2, b_g2, W_ih, W_hh, b_ih, b_hh,
           week_table, minute_table):
    pad = (_N + jnp.arange(_EP - _E, dtype=jnp.int32) % (_NP - _N))
    src3d = jnp.concatenate([edge_index[0], pad]).reshape(_NW, _NCHUNK, _CHUNK)
    dst3d = jnp.concatenate([edge_index[1], pad]).reshape(_NW, _NCHUNK, _CHUNK)

    ones8 = jnp.ones((_CHUNK, 8), jnp.float32)
    zeros8 = jnp.zeros((_ZR, 8), jnp.float32)
    zeros64 = jnp.zeros((_NP, _DSP), jnp.float32)
    dsrc, ddst = _sc_degrees(src3d, dst3d, ones8, zeros8)

    h1pre = _pre1(traffic_h, W_g1, dsrc)
    agg1 = _sc_edge_scatter(h1pre.reshape(_T * _NP, _DSP), src3d, dst3d, zeros64)
    ht = _lstm(traffic_h, W_ih, W_hh,
               b_ih.reshape(1, 4 * _DTM), b_hh.reshape(1, 4 * _DTM))
    h2pre = _mid(agg1, ddst, b_g1.reshape(1, _DSP), W_g2, dsrc)
    agg2 = _sc_edge_scatter(h2pre.reshape(_T * _NP, _DSP), src3d, dst3d, zeros64)
    hs, week_emb, minute_emb = _post(
        agg2, ddst, b_g2.reshape(1, _DSP), local_batch_idx.reshape(_N, 1),
        weeks, minutes, week_table, minute_table)

    return hs, ht, week_emb, minute_emb
